# Initial kernel scaffold; baseline (speedup 1.0000x reference)
import jax
import jax.numpy as jnp
from jax.experimental import pallas as pl


def kernel(feat_user, feat_loc, feat_price, W1, b1, W2, b2, W3, b3, theta):
    raise NotImplementedError("write your pallas kernel here")


# speedup vs baseline: 3.7793x; 3.7793x over previous
import functools

import jax
import jax.numpy as jnp
from jax.experimental import pallas as pl

_B = 16384
_LOC = 64
_K = 64
_EPS = 1e-08
_BLOCK = 2048


_PREC = jax.lax.Precision.HIGHEST


def _fused_body(fu_ref, fl_ref, fp_ref, w1t_ref, b1_ref, w2t_ref, b2_ref,
                w3t_ref, b3_ref, tmuw_ref, tmub_ref, tsdw_ref, tsdb_ref,
                out_ref):
    f32 = jnp.float32
    h = jnp.dot(fu_ref[...], w1t_ref[...], preferred_element_type=f32, precision=_PREC)
    h = h + b1_ref[...]
    h = jnp.where(h >= 0, h, 0.01 * h)
    h = jnp.dot(h, w2t_ref[...], preferred_element_type=f32, precision=_PREC) + b2_ref[...]
    h = jnp.where(h >= 0, h, 0.01 * h)
    z = jnp.dot(h, w3t_ref[...], preferred_element_type=f32, precision=_PREC) + b3_ref[...]

    zmax = jnp.max(z, axis=1, keepdims=True)
    col = jax.lax.broadcasted_iota(jnp.int32, z.shape, 1)
    masked_col = jnp.where(z == zmax, col, _K)
    idx = jnp.min(masked_col, axis=1, keepdims=True)
    onehot = col == idx

    fl = fl_ref[...]
    a_mu = jnp.dot(fl, tmuw_ref[...], preferred_element_type=f32, precision=_PREC) + tmub_ref[...]
    a_sd = jnp.dot(fl, tsdw_ref[...], preferred_element_type=f32, precision=_PREC) + tsdb_ref[...]
    mu = jnp.sum(jnp.where(onehot, a_mu, 0.0), axis=1, keepdims=True)
    d0 = jnp.sum(jnp.where(onehot, a_sd, 0.0), axis=1, keepdims=True)

    delta = jnp.abs(d0) + _EPS
    diff = mu - fp_ref[...]
    pd = (diff * diff) / (delta * delta) * 0.5
    part = jnp.sum(pd - jnp.log(delta), axis=0, keepdims=True)

    @pl.when(pl.program_id(0) == 0)
    def _init():
        out_ref[...] = jnp.zeros((1, 1), jnp.float32)

    out_ref[...] += part


@functools.partial(jax.jit, static_argnames=())
def kernel(feat_user, feat_loc, feat_price, W1, b1, W2, b2, W3, b3, theta):
    n_blocks = _B // _BLOCK
    w1t = W1.T
    w2t = W2.T
    w3t = W3.T
    tmuw = theta[:, 0, :_LOC].T
    tmub = theta[:, 0, _LOC].reshape(1, _K)
    tsdw = theta[:, 1, :_LOC].T
    tsdb = theta[:, 1, _LOC].reshape(1, _K)
    b1r = b1.reshape(1, -1)
    b2r = b2.reshape(1, -1)
    b3r = b3.reshape(1, -1)

    full = lambda shape: pl.BlockSpec(shape, lambda i: (0, 0))
    grid_spec = pl.GridSpec(
        grid=(n_blocks,),
        in_specs=[
            pl.BlockSpec((_BLOCK, 128), lambda i: (i, 0)),
            pl.BlockSpec((_BLOCK, _LOC), lambda i: (i, 0)),
            pl.BlockSpec((_BLOCK, 1), lambda i: (i, 0)),
            full((128, 32)), full((1, 32)),
            full((32, 16)), full((1, 16)),
            full((16, _K)), full((1, _K)),
            full((_LOC, _K)), full((1, _K)),
            full((_LOC, _K)), full((1, _K)),
        ],
        out_specs=pl.BlockSpec((1, 1), lambda i: (0, 0)),
    )
    acc = pl.pallas_call(
        _fused_body,
        grid_spec=grid_spec,
        out_shape=jax.ShapeDtypeStruct((1, 1), jnp.float32),
    )(feat_user, feat_loc, feat_price, w1t, b1r, w2t, b2r, w3t, b3r,
      tmuw, tmub, tsdw, tsdb)
    return acc[0, 0] / _B
